# Initial kernel scaffold; baseline (speedup 1.0000x reference)
import jax
import jax.numpy as jnp
from jax.experimental import pallas as pl


def kernel(node_features, edge_features, senders, receivers, eW1, eb1, eW2, eb2, eln_s, eln_b, nW1, nb1, nW2, nb2, nln_s, nln_b):
    raise NotImplementedError("write your pallas kernel here")


# speedup vs baseline: 4.3815x; 4.3815x over previous
import functools

import jax
import jax.numpy as jnp
from jax import lax
from jax.experimental import pallas as pl
from jax.experimental.pallas import tpu as pltpu
from jax.experimental.pallas import tpu_sc as plsc

_N = 10000
_E = 320000
_D = 128
_STEPS = 8

_NC = 2
_NS = 16
_NW = _NC * _NS
_EPW = _E // _NW
_IB = 80
_KCH = 5
_RCH = _IB * _KCH
_SIB = 40
_SKCH = 5
_SRCH = _SIB * _SKCH
_NPT = 640
_NPT_LAST = _N - 15 * _NPT

_mesh = plsc.VectorSubcoreMesh(core_axis_name="c", subcore_axis_name="s")


@functools.partial(
    pl.kernel,
    mesh=_mesh,
    out_type=(
        jax.ShapeDtypeStruct((_E, _D), jnp.float32),
        jax.ShapeDtypeStruct((_E, _D), jnp.float32),
    ),
    scratch_types=[
        pltpu.VMEM((_EPW,), jnp.int32),
        pltpu.VMEM((_EPW,), jnp.int32),
        pltpu.VMEM((_RCH, _D), jnp.float32),
        pltpu.VMEM((_RCH, _D), jnp.float32),
        pltpu.SemaphoreType.DMA,
        pltpu.SemaphoreType.DMA,
    ],
)
def _sc_gather(nodes_hbm, s_hbm, r_hbm, src_hbm, dst_hbm,
               sidx_v, ridx_v, rows_a, rows_b, sem_a, sem_b):
    wid = lax.axis_index("s") * _NC + lax.axis_index("c")
    base = wid * _EPW
    pltpu.sync_copy(s_hbm.at[pl.ds(base, _EPW)], sidx_v)
    pltpu.sync_copy(r_hbm.at[pl.ds(base, _EPW)], ridx_v)

    @pl.loop(0, _EPW, step=_RCH)
    def _(off):
        hs = [
            pltpu.async_copy(
                nodes_hbm.at[sidx_v.at[pl.ds(off + t * _IB, _IB)]],
                rows_a.at[pl.ds(t * _IB, _IB)], sem_a)
            for t in range(_KCH)
        ]
        hd = [
            pltpu.async_copy(
                nodes_hbm.at[ridx_v.at[pl.ds(off + t * _IB, _IB)]],
                rows_b.at[pl.ds(t * _IB, _IB)], sem_b)
            for t in range(_KCH)
        ]
        for h in hs:
            h.wait()
        pltpu.sync_copy(rows_a, src_hbm.at[pl.ds(base + off, _RCH)])
        for h in hd:
            h.wait()
        pltpu.sync_copy(rows_b, dst_hbm.at[pl.ds(base + off, _RCH)])


@functools.partial(
    pl.kernel,
    mesh=_mesh,
    out_type=jax.ShapeDtypeStruct((_NC, _N, _D), jnp.float32),
    scratch_types=[
        pltpu.VMEM((_SIB,), jnp.int32),
        pltpu.VMEM((_SIB,), jnp.int32),
        pltpu.VMEM((_SIB,), jnp.int32),
        pltpu.VMEM((_SIB,), jnp.int32),
        pltpu.VMEM((_SIB,), jnp.int32),
        pltpu.VMEM((_SIB, _D), jnp.float32),
        pltpu.VMEM((_SIB, _D), jnp.float32),
        pltpu.VMEM((_SIB, _D), jnp.float32),
        pltpu.VMEM((_SIB, _D), jnp.float32),
        pltpu.VMEM((_SIB, _D), jnp.float32),
        pltpu.VMEM_SHARED((_N, _D), jnp.float32),
        pltpu.SemaphoreType.DMA,
        pltpu.SemaphoreType.DMA,
    ],
)
def _sc_scatter(vals_hbm, r_hbm, zeros_hbm, agg_hbm,
                i0, i1, i2, i3, i4, v0, v1, v2, v3, v4, agg_sh, sem_i, sem_v):
    idx_bufs = (i0, i1, i2, i3, i4)
    val_bufs = (v0, v1, v2, v3, v4)
    cid = lax.axis_index("c")
    sid = lax.axis_index("s")
    wid = sid * _NC + cid
    @pl.when(sid < 15)
    def _():
        pltpu.sync_copy(zeros_hbm.at[pl.ds(sid * _NPT, _NPT)],
                        agg_sh.at[pl.ds(sid * _NPT, _NPT)])

    @pl.when(sid == 15)
    def _():
        pltpu.sync_copy(zeros_hbm.at[pl.ds(15 * _NPT, _NPT_LAST)],
                        agg_sh.at[pl.ds(15 * _NPT, _NPT_LAST)])

    plsc.subcore_barrier()

    @pl.loop(0, _EPW, step=_SRCH)
    def _(off):
        b = wid * _EPW + off
        hi = [
            pltpu.async_copy(r_hbm.at[pl.ds(b + t * _SIB, _SIB)],
                             idx_bufs[t], sem_i)
            for t in range(_SKCH)
        ]
        hv = [
            pltpu.async_copy(vals_hbm.at[pl.ds(b + t * _SIB, _SIB)],
                             val_bufs[t], sem_v)
            for t in range(_SKCH)
        ]
        for t in range(_SKCH):
            hi[t].wait()
            hv[t].wait()
            pltpu.sync_copy(val_bufs[t], agg_sh.at[idx_bufs[t]], add=True)

    plsc.subcore_barrier()

    @pl.when(sid < 15)
    def _():
        pltpu.sync_copy(agg_sh.at[pl.ds(sid * _NPT, _NPT)],
                        agg_hbm.at[cid, pl.ds(sid * _NPT, _NPT)])

    @pl.when(sid == 15)
    def _():
        pltpu.sync_copy(agg_sh.at[pl.ds(15 * _NPT, _NPT_LAST)],
                        agg_hbm.at[cid, pl.ds(15 * _NPT, _NPT_LAST)])


def _ln(o, lns, lnb):
    m = jnp.mean(o, axis=1, keepdims=True)
    v = jnp.mean((o - m) * (o - m), axis=1, keepdims=True)
    return (o - m) * lax.rsqrt(v + 1e-5) * lns + lnb


def _edge_block(e_ref, s_ref, d_ref, w1_ref, b1_ref, w2_ref, b2_ref,
                lns_ref, lnb_ref, o_ref):
    x = jnp.concatenate([e_ref[...], s_ref[...], d_ref[...]], axis=1)
    h = jnp.dot(x, w1_ref[...], preferred_element_type=jnp.float32)
    h = jnp.maximum(h + b1_ref[...], 0.0)
    o = jnp.dot(h, w2_ref[...], preferred_element_type=jnp.float32)
    o = o + b2_ref[...]
    o_ref[...] = e_ref[...] + _ln(o, lns_ref[...], lnb_ref[...])


_EB = 4000


def _edge_mlp(edges, src, dst, w1, b1, w2, b2, lns, lnb):
    row = lambda i: (i, 0)
    full = lambda i: (0, 0)
    return pl.pallas_call(
        _edge_block,
        grid=(_E // _EB,),
        in_specs=[
            pl.BlockSpec((_EB, _D), row),
            pl.BlockSpec((_EB, _D), row),
            pl.BlockSpec((_EB, _D), row),
            pl.BlockSpec((3 * _D, _D), full),
            pl.BlockSpec((1, _D), full),
            pl.BlockSpec((_D, _D), full),
            pl.BlockSpec((1, _D), full),
            pl.BlockSpec((1, _D), full),
            pl.BlockSpec((1, _D), full),
        ],
        out_specs=pl.BlockSpec((_EB, _D), row),
        out_shape=jax.ShapeDtypeStruct((_E, _D), jnp.float32),
    )(edges, src, dst, w1, b1, w2, b2, lns, lnb)


def _node_block(n_ref, a0_ref, a1_ref, w1_ref, b1_ref, w2_ref, b2_ref,
                lns_ref, lnb_ref, o_ref):
    agg = a0_ref[...] + a1_ref[...]
    x = jnp.concatenate([n_ref[...], agg], axis=1)
    h = jnp.dot(x, w1_ref[...], preferred_element_type=jnp.float32)
    h = jnp.maximum(h + b1_ref[...], 0.0)
    o = jnp.dot(h, w2_ref[...], preferred_element_type=jnp.float32)
    o = o + b2_ref[...]
    o_ref[...] = n_ref[...] + _ln(o, lns_ref[...], lnb_ref[...])


_NB = 2000


def _node_mlp(nodes, a0, a1, w1, b1, w2, b2, lns, lnb):
    row = lambda i: (i, 0)
    full = lambda i: (0, 0)
    return pl.pallas_call(
        _node_block,
        grid=(_N // _NB,),
        in_specs=[
            pl.BlockSpec((_NB, _D), row),
            pl.BlockSpec((_NB, _D), row),
            pl.BlockSpec((_NB, _D), row),
            pl.BlockSpec((2 * _D, _D), full),
            pl.BlockSpec((1, _D), full),
            pl.BlockSpec((_D, _D), full),
            pl.BlockSpec((1, _D), full),
            pl.BlockSpec((1, _D), full),
            pl.BlockSpec((1, _D), full),
        ],
        out_specs=pl.BlockSpec((_NB, _D), row),
        out_shape=jax.ShapeDtypeStruct((_N, _D), jnp.float32),
    )(nodes, a0, a1, w1, b1, w2, b2, lns, lnb)


def kernel(node_features, edge_features, senders, receivers,
           eW1, eb1, eW2, eb2, eln_s, eln_b,
           nW1, nb1, nW2, nb2, nln_s, nln_b):
    senders = senders.astype(jnp.int32)
    receivers = receivers.astype(jnp.int32)
    zeros = jnp.zeros((_N, _D), jnp.float32)

    nodes = node_features
    edges = edge_features
    r1 = lambda a: a.reshape(1, _D)
    for i in range(_STEPS):
        src, dst = _sc_gather(nodes, senders, receivers)
        e_new = _edge_mlp(edges, src, dst, eW1[i], r1(eb1[i]), eW2[i],
                          r1(eb2[i]), r1(eln_s[i]), r1(eln_b[i]))
        agg = _sc_scatter(e_new, receivers, zeros)
        nodes = _node_mlp(nodes, agg[0], agg[1], nW1[i], r1(nb1[i]), nW2[i],
                          r1(nb2[i]), r1(nln_s[i]), r1(nln_b[i]))
        edges = e_new
    return nodes


# speedup vs baseline: 5.0749x; 1.1583x over previous
import functools

import jax
import jax.numpy as jnp
from jax import lax
from jax.experimental import pallas as pl
from jax.experimental.pallas import tpu as pltpu
from jax.experimental.pallas import tpu_sc as plsc

_N = 10000
_E = 320000
_D = 128
_STEPS = 8

_NC = 2
_NS = 16
_NW = _NC * _NS
_EPW = _E // _NW
_IB = 80
_KCH = 5
_RCH = _IB * _KCH
_SIB = 40
_SKCH = 5
_SRCH = _SIB * _SKCH
_NPT = 640
_NPT_LAST = _N - 15 * _NPT

_mesh = plsc.VectorSubcoreMesh(core_axis_name="c", subcore_axis_name="s")


@functools.partial(
    pl.kernel,
    mesh=_mesh,
    out_type=jax.ShapeDtypeStruct((_E, _D), jnp.float32),
    scratch_types=[
        pltpu.VMEM((_EPW,), jnp.int32),
        pltpu.VMEM((_EPW,), jnp.int32),
        pltpu.VMEM((_RCH, _D), jnp.float32),
        pltpu.VMEM((_RCH, _D), jnp.float32),
        pltpu.SemaphoreType.DMA,
        pltpu.SemaphoreType.DMA,
    ],
)
def _sc_gather_sum(hs_hbm, hr_hbm, s_hbm, r_hbm, out_hbm,
                   sidx_v, ridx_v, rows_a, rows_b, sem_a, sem_b):
    wid = lax.axis_index("s") * _NC + lax.axis_index("c")
    base = wid * _EPW
    pltpu.sync_copy(s_hbm.at[pl.ds(base, _EPW)], sidx_v)
    pltpu.sync_copy(r_hbm.at[pl.ds(base, _EPW)], ridx_v)

    bufs = (rows_a, rows_b)
    sems = (sem_a, sem_b)
    nch = _EPW // _RCH

    def issue_g1(k):
        buf = bufs[k % 2]
        return [
            pltpu.async_copy(
                hs_hbm.at[sidx_v.at[pl.ds(k * _RCH + t * _IB, _IB)]],
                buf.at[pl.ds(t * _IB, _IB)], sems[k % 2])
            for t in range(_KCH)
        ]

    g1 = issue_g1(0)
    for k in range(nch):
        g1_next = issue_g1(k + 1) if k + 1 < nch else None
        buf = bufs[k % 2]
        for h in g1:
            h.wait()
        g2 = [
            pltpu.async_copy(
                hr_hbm.at[ridx_v.at[pl.ds(k * _RCH + t * _IB, _IB)]],
                buf.at[pl.ds(t * _IB, _IB)], sems[k % 2], add=True)
            for t in range(_KCH)
        ]
        for h in g2:
            h.wait()
        pltpu.sync_copy(buf, out_hbm.at[pl.ds(base + k * _RCH, _RCH)])
        g1 = g1_next


@functools.partial(
    pl.kernel,
    mesh=_mesh,
    out_type=jax.ShapeDtypeStruct((_NC, _N, _D), jnp.float32),
    scratch_types=[
        pltpu.VMEM((_SIB,), jnp.int32),
        pltpu.VMEM((_SIB,), jnp.int32),
        pltpu.VMEM((_SIB,), jnp.int32),
        pltpu.VMEM((_SIB,), jnp.int32),
        pltpu.VMEM((_SIB,), jnp.int32),
        pltpu.VMEM((_SIB, _D), jnp.float32),
        pltpu.VMEM((_SIB, _D), jnp.float32),
        pltpu.VMEM((_SIB, _D), jnp.float32),
        pltpu.VMEM((_SIB, _D), jnp.float32),
        pltpu.VMEM((_SIB, _D), jnp.float32),
        pltpu.VMEM_SHARED((_N, _D), jnp.float32),
        pltpu.SemaphoreType.DMA,
        pltpu.SemaphoreType.DMA,
    ],
)
def _sc_scatter(vals_hbm, r_hbm, zeros_hbm, agg_hbm,
                i0, i1, i2, i3, i4, v0, v1, v2, v3, v4, agg_sh, sem_i, sem_v):
    idx_bufs = (i0, i1, i2, i3, i4)
    val_bufs = (v0, v1, v2, v3, v4)
    cid = lax.axis_index("c")
    sid = lax.axis_index("s")
    wid = sid * _NC + cid
    @pl.when(sid < 15)
    def _():
        pltpu.sync_copy(zeros_hbm.at[pl.ds(sid * _NPT, _NPT)],
                        agg_sh.at[pl.ds(sid * _NPT, _NPT)])

    @pl.when(sid == 15)
    def _():
        pltpu.sync_copy(zeros_hbm.at[pl.ds(15 * _NPT, _NPT_LAST)],
                        agg_sh.at[pl.ds(15 * _NPT, _NPT_LAST)])

    plsc.subcore_barrier()

    @pl.loop(0, _EPW, step=_SRCH)
    def _(off):
        b = wid * _EPW + off
        hi = [
            pltpu.async_copy(r_hbm.at[pl.ds(b + t * _SIB, _SIB)],
                             idx_bufs[t], sem_i)
            for t in range(_SKCH)
        ]
        hv = [
            pltpu.async_copy(vals_hbm.at[pl.ds(b + t * _SIB, _SIB)],
                             val_bufs[t], sem_v)
            for t in range(_SKCH)
        ]
        for t in range(_SKCH):
            hi[t].wait()
            hv[t].wait()
            pltpu.sync_copy(val_bufs[t], agg_sh.at[idx_bufs[t]], add=True)

    plsc.subcore_barrier()

    @pl.when(sid < 15)
    def _():
        pltpu.sync_copy(agg_sh.at[pl.ds(sid * _NPT, _NPT)],
                        agg_hbm.at[cid, pl.ds(sid * _NPT, _NPT)])

    @pl.when(sid == 15)
    def _():
        pltpu.sync_copy(agg_sh.at[pl.ds(15 * _NPT, _NPT_LAST)],
                        agg_hbm.at[cid, pl.ds(15 * _NPT, _NPT_LAST)])


def _ln(o, lns, lnb):
    m = jnp.mean(o, axis=1, keepdims=True)
    v = jnp.mean((o - m) * (o - m), axis=1, keepdims=True)
    return (o - m) * lax.rsqrt(v + 1e-5) * lns + lnb


def _proj_block(n_ref, ws_ref, wr_ref, hs_ref, hr_ref):
    n = n_ref[...]
    hs_ref[...] = jnp.dot(n, ws_ref[...], preferred_element_type=jnp.float32)
    hr_ref[...] = jnp.dot(n, wr_ref[...], preferred_element_type=jnp.float32)


_PB = 2000


def _proj(nodes, ws, wr):
    row = lambda i: (i, 0)
    full = lambda i: (0, 0)
    return pl.pallas_call(
        _proj_block,
        grid=(_N // _PB,),
        in_specs=[
            pl.BlockSpec((_PB, _D), row),
            pl.BlockSpec((_D, _D), full),
            pl.BlockSpec((_D, _D), full),
        ],
        out_specs=(
            pl.BlockSpec((_PB, _D), row),
            pl.BlockSpec((_PB, _D), row),
        ),
        out_shape=(
            jax.ShapeDtypeStruct((_N, _D), jnp.float32),
            jax.ShapeDtypeStruct((_N, _D), jnp.float32),
        ),
    )(nodes, ws, wr)


def _edge_block(e_ref, g_ref, w1e_ref, b1_ref, w2_ref, b2_ref,
                lns_ref, lnb_ref, o_ref):
    e = e_ref[...]
    h = jnp.dot(e, w1e_ref[...], preferred_element_type=jnp.float32)
    h = jnp.maximum(h + g_ref[...] + b1_ref[...], 0.0)
    o = jnp.dot(h, w2_ref[...], preferred_element_type=jnp.float32)
    o = o + b2_ref[...]
    o_ref[...] = e + _ln(o, lns_ref[...], lnb_ref[...])


_EB = 4000


def _edge_mlp(edges, gath, w1e, b1, w2, b2, lns, lnb):
    row = lambda i: (i, 0)
    full = lambda i: (0, 0)
    return pl.pallas_call(
        _edge_block,
        grid=(_E // _EB,),
        in_specs=[
            pl.BlockSpec((_EB, _D), row),
            pl.BlockSpec((_EB, _D), row),
            pl.BlockSpec((_D, _D), full),
            pl.BlockSpec((1, _D), full),
            pl.BlockSpec((_D, _D), full),
            pl.BlockSpec((1, _D), full),
            pl.BlockSpec((1, _D), full),
            pl.BlockSpec((1, _D), full),
        ],
        out_specs=pl.BlockSpec((_EB, _D), row),
        out_shape=jax.ShapeDtypeStruct((_E, _D), jnp.float32),
    )(edges, gath, w1e, b1, w2, b2, lns, lnb)


def _node_block(n_ref, a0_ref, a1_ref, w1_ref, b1_ref, w2_ref, b2_ref,
                lns_ref, lnb_ref, o_ref):
    agg = a0_ref[...] + a1_ref[...]
    x = jnp.concatenate([n_ref[...], agg], axis=1)
    h = jnp.dot(x, w1_ref[...], preferred_element_type=jnp.float32)
    h = jnp.maximum(h + b1_ref[...], 0.0)
    o = jnp.dot(h, w2_ref[...], preferred_element_type=jnp.float32)
    o = o + b2_ref[...]
    o_ref[...] = n_ref[...] + _ln(o, lns_ref[...], lnb_ref[...])


_NB = 2000


def _node_mlp(nodes, a0, a1, w1, b1, w2, b2, lns, lnb):
    row = lambda i: (i, 0)
    full = lambda i: (0, 0)
    return pl.pallas_call(
        _node_block,
        grid=(_N // _NB,),
        in_specs=[
            pl.BlockSpec((_NB, _D), row),
            pl.BlockSpec((_NB, _D), row),
            pl.BlockSpec((_NB, _D), row),
            pl.BlockSpec((2 * _D, _D), full),
            pl.BlockSpec((1, _D), full),
            pl.BlockSpec((_D, _D), full),
            pl.BlockSpec((1, _D), full),
            pl.BlockSpec((1, _D), full),
            pl.BlockSpec((1, _D), full),
        ],
        out_specs=pl.BlockSpec((_NB, _D), row),
        out_shape=jax.ShapeDtypeStruct((_N, _D), jnp.float32),
    )(nodes, a0, a1, w1, b1, w2, b2, lns, lnb)


def kernel(node_features, edge_features, senders, receivers,
           eW1, eb1, eW2, eb2, eln_s, eln_b,
           nW1, nb1, nW2, nb2, nln_s, nln_b):
    senders = senders.astype(jnp.int32)
    receivers = receivers.astype(jnp.int32)
    zeros = jnp.zeros((_N, _D), jnp.float32)

    nodes = node_features
    edges = edge_features
    r1 = lambda a: a.reshape(1, _D)
    for i in range(_STEPS):
        w1 = eW1[i]
        hs, hr = _proj(nodes, w1[_D:2 * _D], w1[2 * _D:])
        gath = _sc_gather_sum(hs, hr, senders, receivers)
        e_new = _edge_mlp(edges, gath, w1[:_D], r1(eb1[i]), eW2[i],
                          r1(eb2[i]), r1(eln_s[i]), r1(eln_b[i]))
        agg = _sc_scatter(e_new, receivers, zeros)
        nodes = _node_mlp(nodes, agg[0], agg[1], nW1[i], r1(nb1[i]), nW2[i],
                          r1(nb2[i]), r1(nln_s[i]), r1(nln_b[i]))
        edges = e_new
    return nodes


# speedup vs baseline: 5.1530x; 1.0154x over previous
import functools

import jax
import jax.numpy as jnp
from jax import lax
from jax.experimental import pallas as pl
from jax.experimental.pallas import tpu as pltpu
from jax.experimental.pallas import tpu_sc as plsc

_N = 10000
_E = 320000
_D = 128
_STEPS = 8

_NC = 2
_NS = 16
_NW = _NC * _NS
_EPW = _E // _NW
_IB = 80
_KCH = 5
_RCH = _IB * _KCH
_SIB = 40
_SKCH = 5
_SRCH = _SIB * _SKCH
_NPT = 640
_NPT_LAST = _N - 15 * _NPT

_mesh = plsc.VectorSubcoreMesh(core_axis_name="c", subcore_axis_name="s")


@functools.partial(
    pl.kernel,
    mesh=_mesh,
    out_type=jax.ShapeDtypeStruct((_E, _D), jnp.float32),
    scratch_types=[
        pltpu.VMEM((_EPW,), jnp.int32),
        pltpu.VMEM((_EPW,), jnp.int32),
        pltpu.VMEM((_RCH, _D), jnp.float32),
        pltpu.VMEM((_RCH, _D), jnp.float32),
        pltpu.SemaphoreType.DMA,
        pltpu.SemaphoreType.DMA,
        pltpu.SemaphoreType.DMA,
        pltpu.SemaphoreType.DMA,
    ],
)
def _sc_gather_sum(hs_hbm, hr_hbm, s_hbm, r_hbm, out_hbm,
                   sidx_v, ridx_v, rows_a, rows_b, sem_a, sem_b,
                   wsem_a, wsem_b):
    wid = lax.axis_index("s") * _NC + lax.axis_index("c")
    base = wid * _EPW
    pltpu.sync_copy(s_hbm.at[pl.ds(base, _EPW)], sidx_v)
    pltpu.sync_copy(r_hbm.at[pl.ds(base, _EPW)], ridx_v)

    bufs = (rows_a, rows_b)
    sems = (sem_a, sem_b)
    wsems = (wsem_a, wsem_b)
    nch = _EPW // _RCH

    def issue_g1(k):
        buf = bufs[k % 2]
        return [
            pltpu.async_copy(
                hs_hbm.at[sidx_v.at[pl.ds(k * _RCH + t * _IB, _IB)]],
                buf.at[pl.ds(t * _IB, _IB)], sems[k % 2])
            for t in range(_KCH)
        ]

    g1 = issue_g1(0)
    wb = [None, None]
    for k in range(nch):
        g1_next = issue_g1(k + 1) if k + 1 < nch else None
        buf = bufs[k % 2]
        for h in g1:
            h.wait()
        g2 = [
            pltpu.async_copy(
                hr_hbm.at[ridx_v.at[pl.ds(k * _RCH + t * _IB, _IB)]],
                buf.at[pl.ds(t * _IB, _IB)], sems[k % 2], add=True)
            for t in range(_KCH)
        ]
        for h in g2:
            h.wait()
        if wb[k % 2] is not None:
            wb[k % 2].wait()
        wb[k % 2] = pltpu.async_copy(
            buf, out_hbm.at[pl.ds(base + k * _RCH, _RCH)], wsems[k % 2])
        g1 = g1_next
    for h in wb:
        if h is not None:
            h.wait()


@functools.partial(
    pl.kernel,
    mesh=_mesh,
    out_type=jax.ShapeDtypeStruct((_NC, _N, _D), jnp.float32),
    scratch_types=[
        pltpu.VMEM((_SIB,), jnp.int32),
        pltpu.VMEM((_SIB,), jnp.int32),
        pltpu.VMEM((_SIB,), jnp.int32),
        pltpu.VMEM((_SIB,), jnp.int32),
        pltpu.VMEM((_SIB,), jnp.int32),
        pltpu.VMEM((_SIB, _D), jnp.float32),
        pltpu.VMEM((_SIB, _D), jnp.float32),
        pltpu.VMEM((_SIB, _D), jnp.float32),
        pltpu.VMEM((_SIB, _D), jnp.float32),
        pltpu.VMEM((_SIB, _D), jnp.float32),
        pltpu.VMEM_SHARED((_N, _D), jnp.float32),
        pltpu.SemaphoreType.DMA,
        pltpu.SemaphoreType.DMA,
    ],
)
def _sc_scatter(vals_hbm, r_hbm, zeros_hbm, agg_hbm,
                i0, i1, i2, i3, i4, v0, v1, v2, v3, v4, agg_sh, sem_i, sem_v):
    idx_bufs = (i0, i1, i2, i3, i4)
    val_bufs = (v0, v1, v2, v3, v4)
    cid = lax.axis_index("c")
    sid = lax.axis_index("s")
    wid = sid * _NC + cid
    @pl.when(sid < 15)
    def _():
        pltpu.sync_copy(zeros_hbm.at[pl.ds(sid * _NPT, _NPT)],
                        agg_sh.at[pl.ds(sid * _NPT, _NPT)])

    @pl.when(sid == 15)
    def _():
        pltpu.sync_copy(zeros_hbm.at[pl.ds(15 * _NPT, _NPT_LAST)],
                        agg_sh.at[pl.ds(15 * _NPT, _NPT_LAST)])

    plsc.subcore_barrier()

    @pl.loop(0, _EPW, step=_SRCH)
    def _(off):
        b = wid * _EPW + off
        hi = [
            pltpu.async_copy(r_hbm.at[pl.ds(b + t * _SIB, _SIB)],
                             idx_bufs[t], sem_i)
            for t in range(_SKCH)
        ]
        hv = [
            pltpu.async_copy(vals_hbm.at[pl.ds(b + t * _SIB, _SIB)],
                             val_bufs[t], sem_v)
            for t in range(_SKCH)
        ]
        for t in range(_SKCH):
            hi[t].wait()
            hv[t].wait()
            pltpu.sync_copy(val_bufs[t], agg_sh.at[idx_bufs[t]], add=True)

    plsc.subcore_barrier()

    @pl.when(sid < 15)
    def _():
        pltpu.sync_copy(agg_sh.at[pl.ds(sid * _NPT, _NPT)],
                        agg_hbm.at[cid, pl.ds(sid * _NPT, _NPT)])

    @pl.when(sid == 15)
    def _():
        pltpu.sync_copy(agg_sh.at[pl.ds(15 * _NPT, _NPT_LAST)],
                        agg_hbm.at[cid, pl.ds(15 * _NPT, _NPT_LAST)])


def _ln(o, lns, lnb):
    m = jnp.mean(o, axis=1, keepdims=True)
    v = jnp.mean((o - m) * (o - m), axis=1, keepdims=True)
    return (o - m) * lax.rsqrt(v + 1e-5) * lns + lnb


def _proj_block(n_ref, ws_ref, wr_ref, hs_ref, hr_ref):
    n = n_ref[...]
    hs_ref[...] = jnp.dot(n, ws_ref[...], preferred_element_type=jnp.float32)
    hr_ref[...] = jnp.dot(n, wr_ref[...], preferred_element_type=jnp.float32)


_PB = 2000


def _proj(nodes, ws, wr):
    row = lambda i: (i, 0)
    full = lambda i: (0, 0)
    return pl.pallas_call(
        _proj_block,
        grid=(_N // _PB,),
        in_specs=[
            pl.BlockSpec((_PB, _D), row),
            pl.BlockSpec((_D, _D), full),
            pl.BlockSpec((_D, _D), full),
        ],
        out_specs=(
            pl.BlockSpec((_PB, _D), row),
            pl.BlockSpec((_PB, _D), row),
        ),
        out_shape=(
            jax.ShapeDtypeStruct((_N, _D), jnp.float32),
            jax.ShapeDtypeStruct((_N, _D), jnp.float32),
        ),
    )(nodes, ws, wr)


def _edge_block(e_ref, g_ref, w1e_ref, b1_ref, w2_ref, b2_ref,
                lns_ref, lnb_ref, o_ref):
    e = e_ref[...]
    h = jnp.dot(e, w1e_ref[...], preferred_element_type=jnp.float32)
    h = jnp.maximum(h + g_ref[...] + b1_ref[...], 0.0)
    o = jnp.dot(h, w2_ref[...], preferred_element_type=jnp.float32)
    o = o + b2_ref[...]
    o_ref[...] = e + _ln(o, lns_ref[...], lnb_ref[...])


_EB = 4000


def _edge_mlp(edges, gath, w1e, b1, w2, b2, lns, lnb):
    row = lambda i: (i, 0)
    full = lambda i: (0, 0)
    return pl.pallas_call(
        _edge_block,
        grid=(_E // _EB,),
        in_specs=[
            pl.BlockSpec((_EB, _D), row),
            pl.BlockSpec((_EB, _D), row),
            pl.BlockSpec((_D, _D), full),
            pl.BlockSpec((1, _D), full),
            pl.BlockSpec((_D, _D), full),
            pl.BlockSpec((1, _D), full),
            pl.BlockSpec((1, _D), full),
            pl.BlockSpec((1, _D), full),
        ],
        out_specs=pl.BlockSpec((_EB, _D), row),
        out_shape=jax.ShapeDtypeStruct((_E, _D), jnp.float32),
    )(edges, gath, w1e, b1, w2, b2, lns, lnb)


def _node_block(n_ref, a0_ref, a1_ref, w1_ref, b1_ref, w2_ref, b2_ref,
                lns_ref, lnb_ref, o_ref):
    agg = a0_ref[...] + a1_ref[...]
    x = jnp.concatenate([n_ref[...], agg], axis=1)
    h = jnp.dot(x, w1_ref[...], preferred_element_type=jnp.float32)
    h = jnp.maximum(h + b1_ref[...], 0.0)
    o = jnp.dot(h, w2_ref[...], preferred_element_type=jnp.float32)
    o = o + b2_ref[...]
    o_ref[...] = n_ref[...] + _ln(o, lns_ref[...], lnb_ref[...])


_NB = 2000


def _node_mlp(nodes, a0, a1, w1, b1, w2, b2, lns, lnb):
    row = lambda i: (i, 0)
    full = lambda i: (0, 0)
    return pl.pallas_call(
        _node_block,
        grid=(_N // _NB,),
        in_specs=[
            pl.BlockSpec((_NB, _D), row),
            pl.BlockSpec((_NB, _D), row),
            pl.BlockSpec((_NB, _D), row),
            pl.BlockSpec((2 * _D, _D), full),
            pl.BlockSpec((1, _D), full),
            pl.BlockSpec((_D, _D), full),
            pl.BlockSpec((1, _D), full),
            pl.BlockSpec((1, _D), full),
            pl.BlockSpec((1, _D), full),
        ],
        out_specs=pl.BlockSpec((_NB, _D), row),
        out_shape=jax.ShapeDtypeStruct((_N, _D), jnp.float32),
    )(nodes, a0, a1, w1, b1, w2, b2, lns, lnb)


def kernel(node_features, edge_features, senders, receivers,
           eW1, eb1, eW2, eb2, eln_s, eln_b,
           nW1, nb1, nW2, nb2, nln_s, nln_b):
    senders = senders.astype(jnp.int32)
    receivers = receivers.astype(jnp.int32)
    zeros = jnp.zeros((_N, _D), jnp.float32)

    nodes = node_features
    edges = edge_features
    r1 = lambda a: a.reshape(1, _D)
    for i in range(_STEPS):
        w1 = eW1[i]
        hs, hr = _proj(nodes, w1[_D:2 * _D], w1[2 * _D:])
        gath = _sc_gather_sum(hs, hr, senders, receivers)
        e_new = _edge_mlp(edges, gath, w1[:_D], r1(eb1[i]), eW2[i],
                          r1(eb2[i]), r1(eln_s[i]), r1(eln_b[i]))
        agg = _sc_scatter(e_new, receivers, zeros)
        nodes = _node_mlp(nodes, agg[0], agg[1], nW1[i], r1(nb1[i]), nW2[i],
                          r1(nb2[i]), r1(nln_s[i]), r1(nln_b[i]))
        edges = e_new
    return nodes


# speedup vs baseline: 5.2690x; 1.0225x over previous
import functools

import jax
import jax.numpy as jnp
from jax import lax
from jax.experimental import pallas as pl
from jax.experimental.pallas import tpu as pltpu
from jax.experimental.pallas import tpu_sc as plsc

_N = 10000
_E = 320000
_D = 128
_STEPS = 8

_NC = 2
_NS = 16
_NW = _NC * _NS
_EPW = _E // _NW
_IB = 80
_KCH = 5
_RCH = _IB * _KCH
_SIB = 80
_SKCH = 1
_SRCH = _SIB * _SKCH
_NPT = 640
_NPT_LAST = _N - 15 * _NPT

_mesh = plsc.VectorSubcoreMesh(core_axis_name="c", subcore_axis_name="s")


@functools.partial(
    pl.kernel,
    mesh=_mesh,
    out_type=jax.ShapeDtypeStruct((_E, _D), jnp.float32),
    scratch_types=[
        pltpu.VMEM((_EPW,), jnp.int32),
        pltpu.VMEM((_EPW,), jnp.int32),
        pltpu.VMEM((_RCH, _D), jnp.float32),
        pltpu.VMEM((_RCH, _D), jnp.float32),
        pltpu.SemaphoreType.DMA,
        pltpu.SemaphoreType.DMA,
        pltpu.SemaphoreType.DMA,
        pltpu.SemaphoreType.DMA,
    ],
)
def _sc_gather_sum(hs_hbm, hr_hbm, s_hbm, r_hbm, out_hbm,
                   sidx_v, ridx_v, rows_a, rows_b, sem_a, sem_b,
                   wsem_a, wsem_b):
    wid = lax.axis_index("s") * _NC + lax.axis_index("c")
    base = wid * _EPW
    pltpu.sync_copy(s_hbm.at[pl.ds(base, _EPW)], sidx_v)
    pltpu.sync_copy(r_hbm.at[pl.ds(base, _EPW)], ridx_v)

    bufs = (rows_a, rows_b)
    sems = (sem_a, sem_b)
    wsems = (wsem_a, wsem_b)
    nch = _EPW // _RCH

    def issue_g1(k):
        buf = bufs[k % 2]
        return [
            pltpu.async_copy(
                hs_hbm.at[sidx_v.at[pl.ds(k * _RCH + t * _IB, _IB)]],
                buf.at[pl.ds(t * _IB, _IB)], sems[k % 2])
            for t in range(_KCH)
        ]

    g1 = issue_g1(0)
    wb = [None, None]
    for k in range(nch):
        g1_next = issue_g1(k + 1) if k + 1 < nch else None
        buf = bufs[k % 2]
        for h in g1:
            h.wait()
        g2 = [
            pltpu.async_copy(
                hr_hbm.at[ridx_v.at[pl.ds(k * _RCH + t * _IB, _IB)]],
                buf.at[pl.ds(t * _IB, _IB)], sems[k % 2], add=True)
            for t in range(_KCH)
        ]
        for h in g2:
            h.wait()
        if wb[k % 2] is not None:
            wb[k % 2].wait()
        wb[k % 2] = pltpu.async_copy(
            buf, out_hbm.at[pl.ds(base + k * _RCH, _RCH)], wsems[k % 2])
        g1 = g1_next
    for h in wb:
        if h is not None:
            h.wait()


@functools.partial(
    pl.kernel,
    mesh=_mesh,
    out_type=jax.ShapeDtypeStruct((_NC, _N, _D), jnp.float32),
    scratch_types=[
        pltpu.VMEM((_SRCH,), jnp.int32),
        pltpu.VMEM((_SRCH,), jnp.int32),
        pltpu.VMEM((_SRCH, _D), jnp.float32),
        pltpu.VMEM((_SRCH, _D), jnp.float32),
        pltpu.VMEM_SHARED((_N, _D), jnp.float32),
        pltpu.SemaphoreType.DMA,
        pltpu.SemaphoreType.DMA,
        pltpu.SemaphoreType.DMA,
        pltpu.SemaphoreType.DMA,
    ],
)
def _sc_scatter(vals_hbm, r_hbm, zeros_hbm, agg_hbm,
                ia, ib, va, vb, agg_sh, sem_la, sem_lb, sem_aa, sem_ab):
    idx_bufs = (ia, ib)
    val_bufs = (va, vb)
    lsems = (sem_la, sem_lb)
    asems = (sem_aa, sem_ab)
    cid = lax.axis_index("c")
    sid = lax.axis_index("s")
    wid = sid * _NC + cid
    @pl.when(sid < 15)
    def _():
        pltpu.sync_copy(zeros_hbm.at[pl.ds(sid * _NPT, _NPT)],
                        agg_sh.at[pl.ds(sid * _NPT, _NPT)])

    @pl.when(sid == 15)
    def _():
        pltpu.sync_copy(zeros_hbm.at[pl.ds(15 * _NPT, _NPT_LAST)],
                        agg_sh.at[pl.ds(15 * _NPT, _NPT_LAST)])

    plsc.subcore_barrier()

    base = wid * _EPW
    nch = _EPW // _SRCH

    def issue_loads(k):
        p = k % 2
        return (
            pltpu.async_copy(r_hbm.at[pl.ds(base + k * _SRCH, _SRCH)],
                             idx_bufs[p], lsems[p]),
            pltpu.async_copy(vals_hbm.at[pl.ds(base + k * _SRCH, _SRCH)],
                             val_bufs[p], lsems[p]),
        )

    loads = issue_loads(0)
    adds = [None, None]
    for k in range(nch):
        p = k % 2
        if k + 1 < nch:
            q = (k + 1) % 2
            if adds[q] is not None:
                for h in adds[q]:
                    h.wait()
                adds[q] = None
            loads_next = issue_loads(k + 1)
        else:
            loads_next = None
        for h in loads:
            h.wait()
        adds[p] = [
            pltpu.async_copy(
                val_bufs[p].at[pl.ds(t * _SIB, _SIB)],
                agg_sh.at[idx_bufs[p].at[pl.ds(t * _SIB, _SIB)]],
                asems[p], add=True)
            for t in range(_SKCH)
        ]
        loads = loads_next
    for hs_ in adds:
        if hs_ is not None:
            for h in hs_:
                h.wait()

    plsc.subcore_barrier()

    @pl.when(sid < 15)
    def _():
        pltpu.sync_copy(agg_sh.at[pl.ds(sid * _NPT, _NPT)],
                        agg_hbm.at[cid, pl.ds(sid * _NPT, _NPT)])

    @pl.when(sid == 15)
    def _():
        pltpu.sync_copy(agg_sh.at[pl.ds(15 * _NPT, _NPT_LAST)],
                        agg_hbm.at[cid, pl.ds(15 * _NPT, _NPT_LAST)])


def _ln(o, lns, lnb):
    m = jnp.mean(o, axis=1, keepdims=True)
    v = jnp.mean((o - m) * (o - m), axis=1, keepdims=True)
    return (o - m) * lax.rsqrt(v + 1e-5) * lns + lnb


def _proj_block(n_ref, ws_ref, wr_ref, hs_ref, hr_ref):
    n = n_ref[...]
    hs_ref[...] = jnp.dot(n, ws_ref[...], preferred_element_type=jnp.float32)
    hr_ref[...] = jnp.dot(n, wr_ref[...], preferred_element_type=jnp.float32)


_PB = 2000


def _proj(nodes, ws, wr):
    row = lambda i: (i, 0)
    full = lambda i: (0, 0)
    return pl.pallas_call(
        _proj_block,
        grid=(_N // _PB,),
        in_specs=[
            pl.BlockSpec((_PB, _D), row),
            pl.BlockSpec((_D, _D), full),
            pl.BlockSpec((_D, _D), full),
        ],
        out_specs=(
            pl.BlockSpec((_PB, _D), row),
            pl.BlockSpec((_PB, _D), row),
        ),
        out_shape=(
            jax.ShapeDtypeStruct((_N, _D), jnp.float32),
            jax.ShapeDtypeStruct((_N, _D), jnp.float32),
        ),
    )(nodes, ws, wr)


def _edge_block(e_ref, g_ref, w1e_ref, b1_ref, w2_ref, b2_ref,
                lns_ref, lnb_ref, o_ref):
    e = e_ref[...]
    h = jnp.dot(e, w1e_ref[...], preferred_element_type=jnp.float32)
    h = jnp.maximum(h + g_ref[...] + b1_ref[...], 0.0)
    o = jnp.dot(h, w2_ref[...], preferred_element_type=jnp.float32)
    o = o + b2_ref[...]
    o_ref[...] = e + _ln(o, lns_ref[...], lnb_ref[...])


_EB = 4000


def _edge_mlp(edges, gath, w1e, b1, w2, b2, lns, lnb):
    row = lambda i: (i, 0)
    full = lambda i: (0, 0)
    return pl.pallas_call(
        _edge_block,
        grid=(_E // _EB,),
        in_specs=[
            pl.BlockSpec((_EB, _D), row),
            pl.BlockSpec((_EB, _D), row),
            pl.BlockSpec((_D, _D), full),
            pl.BlockSpec((1, _D), full),
            pl.BlockSpec((_D, _D), full),
            pl.BlockSpec((1, _D), full),
            pl.BlockSpec((1, _D), full),
            pl.BlockSpec((1, _D), full),
        ],
        out_specs=pl.BlockSpec((_EB, _D), row),
        out_shape=jax.ShapeDtypeStruct((_E, _D), jnp.float32),
    )(edges, gath, w1e, b1, w2, b2, lns, lnb)


def _node_block(n_ref, a0_ref, a1_ref, w1_ref, b1_ref, w2_ref, b2_ref,
                lns_ref, lnb_ref, o_ref):
    agg = a0_ref[...] + a1_ref[...]
    x = jnp.concatenate([n_ref[...], agg], axis=1)
    h = jnp.dot(x, w1_ref[...], preferred_element_type=jnp.float32)
    h = jnp.maximum(h + b1_ref[...], 0.0)
    o = jnp.dot(h, w2_ref[...], preferred_element_type=jnp.float32)
    o = o + b2_ref[...]
    o_ref[...] = n_ref[...] + _ln(o, lns_ref[...], lnb_ref[...])


_NB = 2000


def _node_mlp(nodes, a0, a1, w1, b1, w2, b2, lns, lnb):
    row = lambda i: (i, 0)
    full = lambda i: (0, 0)
    return pl.pallas_call(
        _node_block,
        grid=(_N // _NB,),
        in_specs=[
            pl.BlockSpec((_NB, _D), row),
            pl.BlockSpec((_NB, _D), row),
            pl.BlockSpec((_NB, _D), row),
            pl.BlockSpec((2 * _D, _D), full),
            pl.BlockSpec((1, _D), full),
            pl.BlockSpec((_D, _D), full),
            pl.BlockSpec((1, _D), full),
            pl.BlockSpec((1, _D), full),
            pl.BlockSpec((1, _D), full),
        ],
        out_specs=pl.BlockSpec((_NB, _D), row),
        out_shape=jax.ShapeDtypeStruct((_N, _D), jnp.float32),
    )(nodes, a0, a1, w1, b1, w2, b2, lns, lnb)


def kernel(node_features, edge_features, senders, receivers,
           eW1, eb1, eW2, eb2, eln_s, eln_b,
           nW1, nb1, nW2, nb2, nln_s, nln_b):
    senders = senders.astype(jnp.int32)
    receivers = receivers.astype(jnp.int32)
    zeros = jnp.zeros((_N, _D), jnp.float32)

    nodes = node_features
    edges = edge_features
    r1 = lambda a: a.reshape(1, _D)
    for i in range(_STEPS):
        w1 = eW1[i]
        hs, hr = _proj(nodes, w1[_D:2 * _D], w1[2 * _D:])
        gath = _sc_gather_sum(hs, hr, senders, receivers)
        e_new = _edge_mlp(edges, gath, w1[:_D], r1(eb1[i]), eW2[i],
                          r1(eb2[i]), r1(eln_s[i]), r1(eln_b[i]))
        agg = _sc_scatter(e_new, receivers, zeros)
        nodes = _node_mlp(nodes, agg[0], agg[1], nW1[i], r1(nb1[i]), nW2[i],
                          r1(nb2[i]), r1(nln_s[i]), r1(nln_b[i]))
        edges = e_new
    return nodes
